# baseline (speedup 1.0000x reference)
import functools
import math

import jax
import jax.numpy as jnp
from jax import lax
from jax.experimental import pallas as pl
from jax.experimental.pallas import tpu as pltpu
from jax.experimental.pallas import tpu_sc as plsc

_B, _S, _V = 64, 4, 100000
_R = _B * _S
_LS = 0.1
_CONF = 1.0 - _LS
_SV = _LS / (_V - 2)
_ENT = (_V - 2) * _SV * math.log(_SV) + _CONF * math.log(_CONF)

_VB = 2048
_NBLK = (_V + _VB - 1) // _VB


def _sum_kernel(t_ref, x_ref, o_ref, acc_ref):
    j = pl.program_id(0)
    t = t_ref[:, :]
    x = x_ref[:, :]
    wcoef = jnp.where(t == 0, 0.0, -_SV)

    @pl.when(j == 0)
    def _():
        acc_ref[...] = x * wcoef

    @pl.when((j > 0) & (j < _NBLK - 1))
    def _():
        acc_ref[...] = acc_ref[...] + x * wcoef

    @pl.when(j == _NBLK - 1)
    def _():
        col = jax.lax.broadcasted_iota(jnp.int32, (_R, _VB), 1) + j * _VB
        acc_ref[...] = acc_ref[...] + jnp.where(col < _V, x * wcoef, 0.0)
        o_ref[0, 0] = jnp.sum(acc_ref[...])


def _tc_sum(t, x):
    out = pl.pallas_call(
        _sum_kernel,
        grid=(_NBLK,),
        in_specs=[
            pl.BlockSpec((_R, 1), lambda j: (0, 0)),
            pl.BlockSpec((_R, _VB), lambda j: (0, j)),
        ],
        out_specs=pl.BlockSpec(memory_space=pltpu.SMEM),
        out_shape=jax.ShapeDtypeStruct((1, 1), jnp.float32),
        scratch_shapes=[pltpu.VMEM((_R, _VB), jnp.float32)],
        compiler_params=pltpu.CompilerParams(
            dimension_semantics=("arbitrary",),
        ),
    )(t, x)
    return out[0, 0]


def _sc_row_terms(xf, t):
    info = plsc.get_sparse_core_info()
    nc = info.num_cores

    mesh = plsc.VectorSubcoreMesh(core_axis_name="c", subcore_axis_name="s")

    @functools.partial(
        pl.kernel,
        mesh=mesh,
        out_type=jax.ShapeDtypeStruct((_R,), jnp.float32),
        scratch_types=[
            pltpu.VMEM((16,), jnp.int32),
            pltpu.VMEM((16,), jnp.int32),
            pltpu.VMEM((16,), jnp.float32),
            pltpu.VMEM((16,), jnp.float32),
            pltpu.VMEM((16,), jnp.float32),
            pltpu.SemaphoreType.DMA,
        ],
    )
    def k(xf_hbm, t_hbm, out_hbm, t_v, idx_v, valt_v, val0_v, c_v, sem):
        wid = lax.axis_index("s") * nc + lax.axis_index("c")

        @pl.when(wid < _R // 16)
        def _():
            base = wid * 16
            pltpu.sync_copy(t_hbm.at[pl.ds(base, 16)], t_v)
            t = t_v[...]
            rows = base + lax.iota(jnp.int32, 16)
            idx_v[...] = rows * _V + t
            pltpu.async_copy(xf_hbm.at[idx_v], valt_v, sem).wait()
            idx_v[...] = rows * _V
            pltpu.async_copy(xf_hbm.at[idx_v], val0_v, sem).wait()
            c = jnp.where(
                t == 0,
                0.0,
                jnp.float32(_ENT)
                + jnp.float32(_SV) * val0_v[...]
                + jnp.float32(_SV - _CONF) * valt_v[...],
            )
            c_v[...] = c
            pltpu.sync_copy(c_v, out_hbm.at[pl.ds(base, 16)])

    return k(xf, t)


def kernel(output, target, one_hot):
    del one_hot
    x = output.reshape(_R, _V)
    xf = output.reshape(_R * _V)
    t2 = target.reshape(_R, 1)
    tf = target.reshape(_R)
    s1 = _tc_sum(t2, x)
    c = _sc_row_terms(xf, tf)
    return s1 + jnp.sum(c)


# speedup vs baseline: 5.9057x; 5.9057x over previous
import math

import jax
import jax.numpy as jnp
from jax.experimental import pallas as pl
from jax.experimental.pallas import tpu as pltpu

_B, _S, _V = 64, 4, 100000
_LS = 0.1
_CONF = 1.0 - _LS
_SV = _LS / (_V - 2)
_ENT = (_V - 2) * _SV * math.log(_SV) + _CONF * math.log(_CONF)

_VB = 2048
_NBLK = (_V + _VB - 1) // _VB


def _loss_kernel(t_ref, x_ref, o_ref, acc_ref):
    j = pl.program_id(0)
    t = t_ref[...]
    x = x_ref[...]
    lane = jax.lax.broadcasted_iota(jnp.int32, (_B, _S, _VB), 2)
    tl = jnp.where(t == 0, -1, t) - j * _VB
    wrow = jnp.where(t == 0, 0.0, 1.0)
    e = jnp.where(lane == tl, -_CONF, -_SV)
    val = (x * wrow) * e

    @pl.when(j == 0)
    def _():
        acc_ref[...] = jnp.where(lane == 0, 0.0, val)

    @pl.when((j > 0) & (j < _NBLK - 1))
    def _():
        acc_ref[...] = acc_ref[...] + val

    @pl.when(j == _NBLK - 1)
    def _():
        acc_ref[...] = acc_ref[...] + jnp.where(lane >= _V - j * _VB, 0.0, val)
        n_active = jnp.sum(wrow)
        o_ref[0, 0] = jnp.sum(acc_ref[...]) + jnp.float32(_ENT) * n_active


def kernel(output, target, one_hot):
    del one_hot
    t3 = target.reshape(_B, _S, 1)
    out = pl.pallas_call(
        _loss_kernel,
        grid=(_NBLK,),
        in_specs=[
            pl.BlockSpec((_B, _S, 1), lambda j: (0, 0, 0)),
            pl.BlockSpec((_B, _S, _VB), lambda j: (0, 0, j)),
        ],
        out_specs=pl.BlockSpec(memory_space=pltpu.SMEM),
        out_shape=jax.ShapeDtypeStruct((1, 1), jnp.float32),
        scratch_shapes=[pltpu.VMEM((_B, _S, _VB), jnp.float32)],
        compiler_params=pltpu.CompilerParams(
            dimension_semantics=("arbitrary",),
        ),
    )(t3, output)
    return out[0, 0]


# speedup vs baseline: 6.3308x; 1.0720x over previous
import math

import jax
import jax.numpy as jnp
from jax.experimental import pallas as pl
from jax.experimental.pallas import tpu as pltpu

_B, _S, _V = 64, 4, 100000
_LS = 0.1
_CONF = 1.0 - _LS
_SV = _LS / (_V - 2)
_ENT = (_V - 2) * _SV * math.log(_SV) + _CONF * math.log(_CONF)

_VB = 2048
_NSL = _VB // 128
_NBLK = (_V + _VB - 1) // _VB
_LAST_FULL = (_V - (_NBLK - 1) * _VB) // 128
_LAST_REM = _V - (_NBLK - 1) * _VB - _LAST_FULL * 128

def _slices(x, n):
    return [x[:, :, 128 * s:128 * (s + 1)] for s in range(n)]


def _loss_kernel(t_ref, x_ref, o_ref, acc_ref, tacc_ref, tb_ref):
    j = pl.program_id(0)
    lane = jax.lax.broadcasted_iota(jnp.int32, (_B, _S, 128), 2)

    @pl.when(j == 0)
    def _():
        t = t_ref[...]
        tb_ref[...] = jnp.broadcast_to(t, (_B, _S, 128))
        acc_ref[...] = jnp.zeros((_B, _S, 128), jnp.float32)
        tacc_ref[...] = jnp.zeros((_B, _S, 128), jnp.float32)

    tbs = tb_ref[...] - j * _VB

    def accumulate(n_full, mask_rem):
        x = x_ref[...]
        xs = _slices(x, _NSL)
        if mask_rem is not None:
            xs = xs[:n_full + 1]
            xs[n_full] = jnp.where(lane < _LAST_REM, xs[n_full], 0.0)
        ps = xs[0]
        ts = jnp.zeros((_B, _S, 128), jnp.float32)
        for s, xslice in enumerate(xs):
            if s > 0:
                ps = ps + xslice
            m = (lane + s * 128) == tbs
            ts = ts + jnp.where(m, xslice, 0.0)
        acc_ref[...] = acc_ref[...] + ps
        tacc_ref[...] = tacc_ref[...] + ts

    @pl.when(j == 0)
    def _():
        x = x_ref[...]
        xs = _slices(x, _NSL)
        xs[0] = jnp.where(lane == 0, 0.0, xs[0])
        ps = xs[0]
        ts = jnp.zeros((_B, _S, 128), jnp.float32)
        for s in range(_NSL):
            if s > 0:
                ps = ps + xs[s]
            m = (lane + s * 128) == tbs
            ts = ts + jnp.where(m, xs[s], 0.0)
        acc_ref[...] = acc_ref[...] + ps
        tacc_ref[...] = tacc_ref[...] + ts

    @pl.when((j > 0) & (j < _NBLK - 1))
    def _():
        accumulate(_NSL, None)

    @pl.when(j == _NBLK - 1)
    def _():
        accumulate(_LAST_FULL, True)
        t = t_ref[...]
        wrow = jnp.where(t == 0, 0.0, 1.0)
        rsum = jnp.sum(acc_ref[...], axis=2, keepdims=True)
        xt = jnp.sum(tacc_ref[...], axis=2, keepdims=True)
        contrib = wrow * (jnp.float32(_ENT)
                          + jnp.float32(-_SV) * rsum
                          + jnp.float32(_SV - _CONF) * xt)
        o_ref[0, 0] = jnp.sum(contrib)


def kernel(output, target, one_hot):
    del one_hot
    t3 = target.reshape(_B, _S, 1)
    out = pl.pallas_call(
        _loss_kernel,
        grid=(_NBLK,),
        in_specs=[
            pl.BlockSpec((_B, _S, 1), lambda j: (0, 0, 0)),
            pl.BlockSpec((_B, _S, _VB), lambda j: (0, 0, j)),
        ],
        out_specs=pl.BlockSpec(memory_space=pltpu.SMEM),
        out_shape=jax.ShapeDtypeStruct((1, 1), jnp.float32),
        scratch_shapes=[
            pltpu.VMEM((_B, _S, 128), jnp.float32),
            pltpu.VMEM((_B, _S, 128), jnp.float32),
            pltpu.VMEM((_B, _S, 128), jnp.int32),
        ],
        compiler_params=pltpu.CompilerParams(
            dimension_semantics=("arbitrary",),
        ),
    )(t3, output)
    return out[0, 0]


# speedup vs baseline: 6.5875x; 1.0406x over previous
import math

import jax
import jax.numpy as jnp
from jax.experimental import pallas as pl
from jax.experimental.pallas import tpu as pltpu

_B, _S, _V = 64, 4, 100000
_LS = 0.1
_CONF = 1.0 - _LS
_SV = _LS / (_V - 2)
_ENT = (_V - 2) * _SV * math.log(_SV) + _CONF * math.log(_CONF)

_VB = 2048
_NBLK = (_V + _VB - 1) // _VB


def _loss_kernel(t_ref, x_ref, o_ref, acc_ref, tb_ref):
    j = pl.program_id(0)
    lane = jax.lax.broadcasted_iota(jnp.int32, (_B, _S, _VB), 2)

    @pl.when(j == 0)
    def _():
        t = t_ref[...]
        tb_ref[...] = jnp.broadcast_to(t, (_B, _S, _VB))
        x = x_ref[...]
        sel = jnp.where(lane == tb_ref[...], -_CONF, -_SV)
        sel = jnp.where(lane == 0, 0.0, sel)
        acc_ref[...] = x * sel

    @pl.when((j > 0) & (j < _NBLK - 1))
    def _():
        x = x_ref[...]
        gl = lane + j * _VB
        sel = jnp.where(gl == tb_ref[...], -_CONF, -_SV)
        acc_ref[...] = acc_ref[...] + x * sel

    @pl.when(j == _NBLK - 1)
    def _():
        x = x_ref[...]
        gl = lane + j * _VB
        sel = jnp.where(gl == tb_ref[...], -_CONF, -_SV)
        sel = jnp.where(gl >= _V, 0.0, sel)
        acc_ref[...] = acc_ref[...] + jnp.where(gl >= _V, 0.0, x * sel)
        t = t_ref[...]
        wrow = jnp.where(t == 0, 0.0, 1.0)
        rowvals = jnp.sum(acc_ref[...], axis=2, keepdims=True)
        contrib = wrow * (jnp.float32(_ENT) + rowvals)
        o_ref[0, 0] = jnp.sum(contrib)


def kernel(output, target, one_hot):
    del one_hot
    t3 = target.reshape(_B, _S, 1)
    out = pl.pallas_call(
        _loss_kernel,
        grid=(_NBLK,),
        in_specs=[
            pl.BlockSpec((_B, _S, 1), lambda j: (0, 0, 0)),
            pl.BlockSpec((_B, _S, _VB), lambda j: (0, 0, j)),
        ],
        out_specs=pl.BlockSpec(memory_space=pltpu.SMEM),
        out_shape=jax.ShapeDtypeStruct((1, 1), jnp.float32),
        scratch_shapes=[
            pltpu.VMEM((_B, _S, _VB), jnp.float32),
            pltpu.VMEM((_B, _S, _VB), jnp.int32),
        ],
        compiler_params=pltpu.CompilerParams(
            dimension_semantics=("arbitrary",),
        ),
    )(t3, output)
    return out[0, 0]


# speedup vs baseline: 7.5310x; 1.1432x over previous
import math

import jax
import jax.numpy as jnp
from jax.experimental import pallas as pl
from jax.experimental.pallas import tpu as pltpu

_B, _S, _V = 64, 4, 100000
_LS = 0.1
_CONF = 1.0 - _LS
_SV = _LS / (_V - 2)
_ENT = (_V - 2) * _SV * math.log(_SV) + _CONF * math.log(_CONF)

_VB = 8192
_NBLK = (_V + _VB - 1) // _VB


def _loss_kernel(t_ref, x_ref, o_ref, acc_ref, tb_ref):
    j = pl.program_id(0)
    lane = jax.lax.broadcasted_iota(jnp.int32, (_B, _S, _VB), 2)

    @pl.when(j == 0)
    def _():
        t = t_ref[...]
        tb_ref[...] = jnp.broadcast_to(t, (_B, _S, _VB))
        x = x_ref[...]
        sel = jnp.where(lane == tb_ref[...], -_CONF, -_SV)
        sel = jnp.where(lane == 0, 0.0, sel)
        acc_ref[...] = x * sel

    @pl.when((j > 0) & (j < _NBLK - 1))
    def _():
        x = x_ref[...]
        gl = lane + j * _VB
        sel = jnp.where(gl == tb_ref[...], -_CONF, -_SV)
        acc_ref[...] = acc_ref[...] + x * sel

    @pl.when(j == _NBLK - 1)
    def _():
        x = x_ref[...]
        gl = lane + j * _VB
        sel = jnp.where(gl == tb_ref[...], -_CONF, -_SV)
        sel = jnp.where(gl >= _V, 0.0, sel)
        acc_ref[...] = acc_ref[...] + jnp.where(gl >= _V, 0.0, x * sel)
        t = t_ref[...]
        wrow = jnp.where(t == 0, 0.0, 1.0)
        rowvals = jnp.sum(acc_ref[...], axis=2, keepdims=True)
        contrib = wrow * (jnp.float32(_ENT) + rowvals)
        o_ref[0, 0] = jnp.sum(contrib)


def kernel(output, target, one_hot):
    del one_hot
    t3 = target.reshape(_B, _S, 1)
    out = pl.pallas_call(
        _loss_kernel,
        grid=(_NBLK,),
        in_specs=[
            pl.BlockSpec((_B, _S, 1), lambda j: (0, 0, 0)),
            pl.BlockSpec((_B, _S, _VB), lambda j: (0, 0, j)),
        ],
        out_specs=pl.BlockSpec(memory_space=pltpu.SMEM),
        out_shape=jax.ShapeDtypeStruct((1, 1), jnp.float32),
        scratch_shapes=[
            pltpu.VMEM((_B, _S, _VB), jnp.float32),
            pltpu.VMEM((_B, _S, _VB), jnp.int32),
        ],
        compiler_params=pltpu.CompilerParams(
            dimension_semantics=("arbitrary",),
        ),
    )(t3, output)
    return out[0, 0]
